# baseline (speedup 1.0000x reference)
import jax
import jax.numpy as jnp
from jax import lax
from jax.experimental import pallas as pl
from jax.experimental.pallas import tpu as pltpu
from jax.experimental.pallas import tpu_sc as plsc

NUM_POINTS = 2_000_000
N_NEW = 1_000_000
N_FEAT = 1000
D_FEAT = 512

NC = 2
NS = 16
NW = NC * NS

PTS_NEW = N_NEW * 3
PTS_TOT = NUM_POINTS * 3
FEAT_TOT = N_FEAT * D_FEAT

PW = 24
PCHUNK = PTS_NEW // PW
LW = NW - PW
LCHUNK = N_NEW // LW
FCHUNK = FEAT_TOT // NW

CH = 25_000
NSUB = PCHUNK // CH

assert PCHUNK * PW == PTS_NEW and PCHUNK % 8 == 0
assert LCHUNK * LW == N_NEW and LCHUNK == PCHUNK
assert FCHUNK * NW == FEAT_TOT and FCHUNK % 8 == 0
assert NSUB * CH == PCHUNK and CH % 8 == 0 and FCHUNK <= CH


def _staged_pipeline(subs, bufs, rsem, wsem):
    n = len(subs)
    reads, writes = [], []
    for i, (src, dst, off) in enumerate(subs):
        reads.append(pltpu.make_async_copy(
            src.at[pl.ds(off, CH)], bufs[i % 2], rsem))
        writes.append(pltpu.make_async_copy(
            bufs[i % 2], dst.at[pl.ds(off, CH)], wsem))
    reads[0].start()
    for i in range(n):
        reads[i].wait()
        writes[i].start()
        if i + 1 < n:
            if i >= 1:
                writes[i - 1].wait()
            reads[i + 1].start()
    if n >= 2:
        writes[n - 2].wait()
    writes[n - 1].wait()


def _copy_body(np_ref, pose_ref, nl_ref, feat_ref, pbuf_ref, lbuf_ref,
               out_p, out_pose, out_l, out_f,
               fbuf0, fbuf1, ibuf0, ibuf1, rsem, wsem):
    wid = lax.axis_index("s") * NC + lax.axis_index("c")

    fb = pl.multiple_of(wid * FCHUNK, 8)
    feat_rd = pltpu.make_async_copy(
        feat_ref.at[pl.ds(fb, FCHUNK)], fbuf0.at[pl.ds(0, FCHUNK)], rsem)
    feat_wr = pltpu.make_async_copy(
        fbuf0.at[pl.ds(0, FCHUNK)], out_f.at[pl.ds(fb, FCHUNK)], wsem)
    feat_rd.start()
    feat_rd.wait()
    feat_wr.start()
    feat_wr.wait()

    @pl.when(wid == 0)
    def _pose():
        pose_rd = pltpu.make_async_copy(pose_ref, fbuf0.at[pl.ds(0, 16)], rsem)
        pose_wr = pltpu.make_async_copy(fbuf0.at[pl.ds(0, 16)], out_pose, wsem)
        pose_rd.start()
        pose_rd.wait()
        pose_wr.start()
        pose_wr.wait()

    @pl.when(wid < PW)
    def _points():
        b = pl.multiple_of(wid * PCHUNK, 8)
        t = pl.multiple_of(PTS_NEW + wid * PCHUNK, 8)
        subs = ([(np_ref, out_p, b + j * CH) for j in range(NSUB)]
                + [(pbuf_ref, out_p, t + j * CH) for j in range(NSUB)])
        _staged_pipeline(subs, (fbuf0, fbuf1), rsem, wsem)

    @pl.when(wid >= PW)
    def _labels():
        w = wid - PW
        b = pl.multiple_of(w * LCHUNK, 8)
        t = pl.multiple_of(N_NEW + w * LCHUNK, 8)
        subs = ([(nl_ref, out_l, b + j * CH) for j in range(NSUB)]
                + [(lbuf_ref, out_l, t + j * CH) for j in range(NSUB)])
        _staged_pipeline(subs, (ibuf0, ibuf1), rsem, wsem)


@jax.jit
def _scatter_copy(np_flat, pose_flat, new_labels, feat_flat, pbuf_flat, lbuf):
    run = pl.kernel(
        _copy_body,
        out_type=(
            jax.ShapeDtypeStruct((PTS_TOT,), jnp.float32),
            jax.ShapeDtypeStruct((16,), jnp.float32),
            jax.ShapeDtypeStruct((NUM_POINTS,), jnp.int32),
            jax.ShapeDtypeStruct((FEAT_TOT,), jnp.float32),
        ),
        mesh=plsc.VectorSubcoreMesh(
            core_axis_name="c", subcore_axis_name="s"),
        scratch_types=[
            pltpu.VMEM((CH,), jnp.float32),
            pltpu.VMEM((CH,), jnp.float32),
            pltpu.VMEM((CH,), jnp.int32),
            pltpu.VMEM((CH,), jnp.int32),
            pltpu.SemaphoreType.DMA,
            pltpu.SemaphoreType.DMA,
        ],
    )
    return run(np_flat, pose_flat, new_labels, feat_flat, pbuf_flat, lbuf)


def kernel(new_points, pose, new_point_label, new_label_feature,
           points_buf, points_label_buf, label_feature_buf, pose_buf):
    del label_feature_buf, pose_buf
    out_p, out_pose, out_l, out_f = _scatter_copy(
        new_points.reshape(PTS_NEW),
        pose.reshape(16),
        new_point_label,
        new_label_feature.reshape(FEAT_TOT),
        points_buf.reshape(PTS_TOT),
        points_label_buf,
    )
    return (out_p.reshape(NUM_POINTS, 3), out_pose.reshape(4, 4), out_l,
            out_f.reshape(N_FEAT, D_FEAT))
